# baseline (speedup 1.0000x reference)
import functools

import jax
import jax.numpy as jnp
from jax.experimental import pallas as pl
from jax.experimental.pallas import tpu as pltpu

VOCAB = 100000
D_MODEL = 1024
RANK = 16
V1 = 317
V2 = 316
D1 = 32
D2 = 32
VPAD = 320

TOK_BLK = 256


def _tt_body(ids_ref, a_ref, b_ref, ph_ref, out_ref):
    ids = ids_ref[0]
    idx1 = jnp.clip(ids // V2, 0, V1 - 1)
    idx2 = jnp.clip(ids % V2, 0, V2 - 1)

    iota = jax.lax.broadcasted_iota(jnp.int32, (TOK_BLK, VPAD), 1)
    oh1 = jnp.where(iota == idx1, 1.0, 0.0).astype(jnp.float32)
    oh2 = jnp.where(iota == idx2, 1.0, 0.0).astype(jnp.float32)

    c1 = jnp.dot(oh1, a_ref[...], preferred_element_type=jnp.float32)
    c2 = jnp.dot(oh2, b_ref[...], preferred_element_type=jnp.float32)

    pm = jnp.cos(ph_ref[...])
    c1 = jnp.clip(c1 * pm, -10.0, 10.0)
    c2 = jnp.clip(c2, -10.0, 10.0)

    for d in range(D1):
        acc = c1[:, d : d + 1] * c2[:, 0:D2]
        for r in range(1, RANK):
            acc = acc + c1[:, 32 * r + d : 32 * r + d + 1] * c2[:, 32 * r : 32 * r + D2]
        out_ref[:, D2 * d : D2 * (d + 1)] = acc


@jax.jit
def kernel(input_ids, core1, core2, phase_shift):
    b, l = input_ids.shape
    n_tok = b * l
    n_blk = n_tok // TOK_BLK

    ids3 = input_ids.reshape(n_blk, TOK_BLK, 1)
    a = jnp.pad(core1.reshape(V1, RANK * D1), ((0, VPAD - V1), (0, 0)))
    bb = jnp.pad(core2.reshape(V2, RANK * D2), ((0, VPAD - V2), (0, 0)))
    ph = jnp.repeat(phase_shift, D1).reshape(1, RANK * D1)

    out = pl.pallas_call(
        _tt_body,
        grid=(n_blk,),
        in_specs=[
            pl.BlockSpec((1, TOK_BLK, 1), lambda i: (i, 0, 0)),
            pl.BlockSpec((VPAD, RANK * D1), lambda i: (0, 0)),
            pl.BlockSpec((VPAD, RANK * D2), lambda i: (0, 0)),
            pl.BlockSpec((1, RANK * D1), lambda i: (0, 0)),
        ],
        out_specs=pl.BlockSpec((TOK_BLK, D_MODEL), lambda i: (i, 0)),
        out_shape=jax.ShapeDtypeStruct((n_tok, D_MODEL), jnp.float32),
        compiler_params=pltpu.CompilerParams(
            dimension_semantics=("arbitrary",),
        ),
    )(ids3, a, bb, ph)
    return out.reshape(b, l, D_MODEL)


# speedup vs baseline: 19.7111x; 19.7111x over previous
import jax
import jax.numpy as jnp
from jax.experimental import pallas as pl
from jax.experimental.pallas import tpu as pltpu

VOCAB = 100000
D_MODEL = 1024
RANK = 16
V1 = 317
V2 = 316
D1 = 32
D2 = 32
VPAD = 320
KD = RANK * D1

TOK_BLK = 512


def _tt_body(ids_ref, at_ref, bt_ref, ph_ref, out_ref, ah_ref, al_ref, bh_ref, bl_ref):
    i = pl.program_id(0)

    @pl.when(i == 0)
    def _split_tables():
        a = at_ref[...]
        ah = a.astype(jnp.bfloat16)
        ah_ref[...] = ah
        al_ref[...] = (a - ah.astype(jnp.float32)).astype(jnp.bfloat16)
        b = bt_ref[...]
        bh = b.astype(jnp.bfloat16)
        bh_ref[...] = bh
        bl_ref[...] = (b - bh.astype(jnp.float32)).astype(jnp.bfloat16)

    ids = ids_ref[0]
    idx1 = jnp.clip(ids // V2, 0, V1 - 1)
    idx2 = jnp.clip(ids % V2, 0, V2 - 1)

    iota0 = jax.lax.broadcasted_iota(jnp.int32, (VPAD, TOK_BLK), 0)
    oh1 = (iota0 == idx1).astype(jnp.bfloat16)
    oh2 = (iota0 == idx2).astype(jnp.bfloat16)

    c1 = jnp.dot(ah_ref[...], oh1, preferred_element_type=jnp.float32)
    c1 = c1 + jnp.dot(al_ref[...], oh1, preferred_element_type=jnp.float32)
    c2 = jnp.dot(bh_ref[...], oh2, preferred_element_type=jnp.float32)
    c2 = c2 + jnp.dot(bl_ref[...], oh2, preferred_element_type=jnp.float32)

    pm = jnp.cos(ph_ref[...])
    c1 = jnp.clip(c1 * pm, -10.0, 10.0)
    c2 = jnp.clip(c2, -10.0, 10.0)

    accs = []
    for d in range(D1):
        acc = c1[d : d + 1, :] * c2[0:D2, :]
        for r in range(1, RANK):
            acc = acc + c1[32 * r + d : 32 * r + d + 1, :] * c2[32 * r : 32 * r + D2, :]
        accs.append(acc)
    out_t = jnp.concatenate(accs, axis=0)
    out_ref[...] = out_t.T


@jax.jit
def kernel(input_ids, core1, core2, phase_shift):
    b, l = input_ids.shape
    n_tok = b * l
    n_blk = n_tok // TOK_BLK

    ids3 = input_ids.reshape(n_blk, 1, TOK_BLK)
    a_t = jnp.pad(core1.reshape(V1, KD), ((0, VPAD - V1), (0, 0))).T
    b_t = jnp.pad(core2.reshape(V2, KD), ((0, VPAD - V2), (0, 0))).T
    ph = jnp.repeat(phase_shift, D1).reshape(KD, 1)

    out = pl.pallas_call(
        _tt_body,
        grid=(n_blk,),
        in_specs=[
            pl.BlockSpec((1, 1, TOK_BLK), lambda i: (i, 0, 0)),
            pl.BlockSpec((KD, VPAD), lambda i: (0, 0)),
            pl.BlockSpec((KD, VPAD), lambda i: (0, 0)),
            pl.BlockSpec((KD, 1), lambda i: (0, 0)),
        ],
        out_specs=pl.BlockSpec((TOK_BLK, D_MODEL), lambda i: (i, 0)),
        out_shape=jax.ShapeDtypeStruct((n_tok, D_MODEL), jnp.float32),
        scratch_shapes=[
            pltpu.VMEM((KD, VPAD), jnp.bfloat16),
            pltpu.VMEM((KD, VPAD), jnp.bfloat16),
            pltpu.VMEM((KD, VPAD), jnp.bfloat16),
            pltpu.VMEM((KD, VPAD), jnp.bfloat16),
        ],
        compiler_params=pltpu.CompilerParams(
            dimension_semantics=("arbitrary",),
        ),
    )(ids3, a_t, b_t, ph)
    return out.reshape(b, l, D_MODEL)


# speedup vs baseline: 21.4149x; 1.0864x over previous
import jax
import jax.numpy as jnp
from jax.experimental import pallas as pl
from jax.experimental.pallas import tpu as pltpu

VOCAB = 100000
D_MODEL = 1024
RANK = 16
V1 = 317
V2 = 316
D1 = 32
D2 = 32
VPAD = 320
KD = RANK * D1

TOK_BLK = 512


def _tt_body(ids_ref, at_ref, bt_ref, ph_ref, out_ref, ah_ref, al_ref, bh_ref, bl_ref):
    i = pl.program_id(0)

    @pl.when(i == 0)
    def _split_tables():
        pm = jnp.cos(ph_ref[...])
        a = jnp.clip(at_ref[...] * pm, -10.0, 10.0)
        ah = a.astype(jnp.bfloat16)
        ah_ref[...] = ah
        al_ref[...] = (a - ah.astype(jnp.float32)).astype(jnp.bfloat16)
        b = jnp.clip(bt_ref[...], -10.0, 10.0)
        bh = b.astype(jnp.bfloat16)
        bh_ref[...] = bh
        bl_ref[...] = (b - bh.astype(jnp.float32)).astype(jnp.bfloat16)

    ids = ids_ref[0]
    idx1 = jnp.clip(ids // V2, 0, V1 - 1)
    idx2 = jnp.clip(ids % V2, 0, V2 - 1)

    iota0 = jax.lax.broadcasted_iota(jnp.int32, (VPAD, TOK_BLK), 0)
    oh1 = (iota0 == idx1).astype(jnp.bfloat16)
    oh2 = (iota0 == idx2).astype(jnp.bfloat16)

    c1 = jnp.dot(ah_ref[...], oh1, preferred_element_type=jnp.float32)
    c1 = c1 + jnp.dot(al_ref[...], oh1, preferred_element_type=jnp.float32)
    c2 = jnp.dot(bh_ref[...], oh2, preferred_element_type=jnp.float32)
    c2 = c2 + jnp.dot(bl_ref[...], oh2, preferred_element_type=jnp.float32)

    accs = []
    for d in range(D1):
        acc = c1[d : d + 1, :] * c2[0:D2, :]
        for r in range(1, RANK):
            acc = acc + c1[32 * r + d : 32 * r + d + 1, :] * c2[32 * r : 32 * r + D2, :]
        accs.append(acc)
    out_t = jnp.concatenate(accs, axis=0)
    out_ref[...] = out_t.T


@jax.jit
def kernel(input_ids, core1, core2, phase_shift):
    b, l = input_ids.shape
    n_tok = b * l
    n_blk = n_tok // TOK_BLK

    ids3 = input_ids.reshape(n_blk, 1, TOK_BLK)
    a_t = jnp.pad(core1.reshape(V1, KD), ((0, VPAD - V1), (0, 0))).T
    b_t = jnp.pad(core2.reshape(V2, KD), ((0, VPAD - V2), (0, 0))).T
    ph = jnp.repeat(phase_shift, D1).reshape(KD, 1)

    out = pl.pallas_call(
        _tt_body,
        grid=(n_blk,),
        in_specs=[
            pl.BlockSpec((1, 1, TOK_BLK), lambda i: (i, 0, 0)),
            pl.BlockSpec((KD, VPAD), lambda i: (0, 0)),
            pl.BlockSpec((KD, VPAD), lambda i: (0, 0)),
            pl.BlockSpec((KD, 1), lambda i: (0, 0)),
        ],
        out_specs=pl.BlockSpec((TOK_BLK, D_MODEL), lambda i: (i, 0)),
        out_shape=jax.ShapeDtypeStruct((n_tok, D_MODEL), jnp.float32),
        scratch_shapes=[
            pltpu.VMEM((KD, VPAD), jnp.bfloat16),
            pltpu.VMEM((KD, VPAD), jnp.bfloat16),
            pltpu.VMEM((KD, VPAD), jnp.bfloat16),
            pltpu.VMEM((KD, VPAD), jnp.bfloat16),
        ],
        compiler_params=pltpu.CompilerParams(
            dimension_semantics=("arbitrary",),
        ),
    )(ids3, a_t, b_t, ph)
    return out.reshape(b, l, D_MODEL)
